# baseline (speedup 1.0000x reference)
import functools

import jax
import jax.numpy as jnp
from jax import lax
from jax.experimental import pallas as pl
from jax.experimental.pallas import tpu as pltpu
from jax.experimental.pallas import tpu_sc as plsc

CORPUS = 1_000_000
D = 64
NI, NJ = 16384, 50
B = NI * NJ
NW = 32
NIT = NI // 128
NBLK = NJ * NIT
BPW = NBLK // NW

_mesh = plsc.VectorSubcoreMesh(core_axis_name="c", subcore_axis_name="s")


@functools.partial(
    pl.kernel,
    out_type=jax.ShapeDtypeStruct((NJ, 8, 128, 8, 128), jnp.float32),
    mesh=_mesh,
    scratch_types=[
        pltpu.VMEM((2, 128), jnp.int32),
        pltpu.VMEM((2, 128, D), jnp.float32),
        pltpu.VMEM((2, 8, 8, 128), jnp.float32),
        pltpu.SemaphoreType.DMA,
        pltpu.SemaphoreType.DMA,
        pltpu.SemaphoreType.DMA,
        pltpu.SemaphoreType.DMA,
        pltpu.SemaphoreType.DMA,
        pltpu.SemaphoreType.DMA,
    ],
    compiler_params=pltpu.CompilerParams(
        use_tc_tiling_on_sc=False, needs_layout_passes=False
    ),
)
def _emb_lookup(idx_hbm, table_hbm, out_hbm, idx_v, rows_v, obuf_v,
                gi0, gi1, gr0, gr1, po0, po1):
    wid = lax.axis_index("s") * 2 + lax.axis_index("c")
    isem = (gi0, gi1)
    gsem = (gr0, gr1)
    osem = (po0, po1)

    def block_id(t):
        return wid * BPW + t

    def fetch(t, s):
        b = block_id(t)
        n0 = (b // NIT) * NI + (b % NIT) * 128
        pltpu.async_copy(idx_hbm.at[pl.ds(n0, 128)], idx_v.at[s], isem[s])

    def fetch_wait(t, s):
        b = block_id(t)
        n0 = (b // NIT) * NI + (b % NIT) * 128
        pltpu.make_async_copy(
            idx_hbm.at[pl.ds(n0, 128)], idx_v.at[s], isem[s]
        ).wait()
        pltpu.async_copy(table_hbm.at[idx_v.at[s]], rows_v.at[s], gsem[s])

    def gather_wait(t, s):
        pltpu.make_async_copy(
            table_hbm.at[idx_v.at[s]], rows_v.at[s], gsem[s]
        ).wait()

    def put(t, s):
        b = block_id(t)
        pltpu.async_copy(
            obuf_v.at[s], out_hbm.at[b // NIT, :, b % NIT], osem[s]
        )

    def put_wait(t, s):
        b = block_id(t)
        pltpu.make_async_copy(
            obuf_v.at[s], out_hbm.at[b // NIT, :, b % NIT], osem[s]
        ).wait()

    def transpose(s):
        rows = rows_v.at[s]
        obuf = obuf_v.at[s]

        def kbody(k, _):
            col = jax.lax.broadcast(k, (16,))

            def lbody(l, _):
                il = l * 16
                rvec = plsc.load_gather(
                    rows, [il + lax.iota(jnp.int32, 16), col]
                )
                obuf[k // 8, k % 8, pl.ds(il, 16)] = rvec
                return 0

            lax.fori_loop(0, 8, lbody, 0, unroll=True)
            return 0

        lax.fori_loop(0, D, kbody, 0, unroll=8)

    fetch(0, 0)
    fetch(1, 1)
    fetch_wait(0, 0)

    def body(p, _):
        a = 2 * p
        fetch_wait(a + 1, 1)

        gather_wait(a, 0)
        @pl.when(p >= 1)
        def _():
            put_wait(a - 2, 0)
        transpose(0)
        @pl.when(a + 2 < BPW)
        def _():
            fetch(a + 2, 0)
        put(a, 0)
        @pl.when(a + 2 < BPW)
        def _():
            fetch_wait(a + 2, 0)

        gather_wait(a + 1, 1)
        @pl.when(p >= 1)
        def _():
            put_wait(a - 1, 1)
        transpose(1)
        @pl.when(a + 3 < BPW)
        def _():
            fetch(a + 3, 1)
        put(a + 1, 1)
        return 0

    lax.fori_loop(0, BPW // 2, body, 0)
    put_wait(BPW - 2, 0)
    put_wait(BPW - 1, 1)


def kernel(idx, table):
    idx_t = jnp.transpose(idx).reshape(-1).astype(jnp.int32)
    out_v = _emb_lookup(idx_t, table)
    return out_v.transpose(2, 4, 0, 1, 3).reshape(NI, NJ, D)


# speedup vs baseline: 1.3964x; 1.3964x over previous
import functools

import jax
import jax.numpy as jnp
from jax import lax
from jax.experimental import pallas as pl
from jax.experimental.pallas import tpu as pltpu
from jax.experimental.pallas import tpu_sc as plsc

CORPUS = 1_000_000
D = 64
NI, NJ = 16384, 50
B = NI * NJ
NW = 32
NIT = NI // 128
NBLK = NJ * NIT
BPW = NBLK // NW

_mesh = plsc.VectorSubcoreMesh(core_axis_name="c", subcore_axis_name="s")


@functools.partial(
    pl.kernel,
    out_type=jax.ShapeDtypeStruct((NJ, 8, 128, 8, 128), jnp.float32),
    mesh=_mesh,
    scratch_types=[
        pltpu.VMEM((2, 128), jnp.int32),
        pltpu.VMEM((2, 128, D), jnp.float32),
        pltpu.VMEM((2, 8, 8, 128), jnp.float32),
        pltpu.SemaphoreType.DMA,
        pltpu.SemaphoreType.DMA,
        pltpu.SemaphoreType.DMA,
        pltpu.SemaphoreType.DMA,
        pltpu.SemaphoreType.DMA,
        pltpu.SemaphoreType.DMA,
    ],
    compiler_params=pltpu.CompilerParams(
        use_tc_tiling_on_sc=False, needs_layout_passes=False
    ),
)
def _emb_lookup(idx_hbm, table_hbm, out_hbm, idx_v, rows_v, obuf_v,
                gi0, gi1, gr0, gr1, po0, po1):
    wid = lax.axis_index("s") * 2 + lax.axis_index("c")
    isem = (gi0, gi1)
    gsem = (gr0, gr1)
    osem = (po0, po1)

    def block_id(t):
        return wid * BPW + t

    def fetch(t, s):
        b = block_id(t)
        n0 = (b // NIT) * NI + (b % NIT) * 128
        pltpu.async_copy(idx_hbm.at[pl.ds(n0, 128)], idx_v.at[s], isem[s])

    def fetch_wait(t, s):
        b = block_id(t)
        n0 = (b // NIT) * NI + (b % NIT) * 128
        pltpu.make_async_copy(
            idx_hbm.at[pl.ds(n0, 128)], idx_v.at[s], isem[s]
        ).wait()
        pltpu.async_copy(table_hbm.at[idx_v.at[s]], rows_v.at[s], gsem[s])

    def gather_wait(t, s):
        pltpu.make_async_copy(
            table_hbm.at[idx_v.at[s]], rows_v.at[s], gsem[s]
        ).wait()

    def put(t, s):
        b = block_id(t)
        pltpu.async_copy(
            obuf_v.at[s], out_hbm.at[b // NIT, :, b % NIT], osem[s]
        )

    def put_wait(t, s):
        b = block_id(t)
        pltpu.make_async_copy(
            obuf_v.at[s], out_hbm.at[b // NIT, :, b % NIT], osem[s]
        ).wait()

    def transpose(s):
        rows = rows_v.at[s]
        obuf = obuf_v.at[s]
        lane = lax.iota(jnp.int32, 16)
        rbs = [lane + (16 * q) for q in range(8)]

        def kbody(k):
            col = jax.lax.broadcast(k, (16,))
            for q in range(8):
                obuf[k // 8, k % 8, pl.ds(16 * q, 16)] = plsc.load_gather(
                    rows, [rbs[q], col]
                )

        plsc.parallel_loop(0, D, 1, unroll=8)(kbody)

    fetch(0, 0)
    fetch(1, 1)
    fetch_wait(0, 0)

    def body(p, _):
        a = 2 * p
        fetch_wait(a + 1, 1)

        gather_wait(a, 0)
        @pl.when(p >= 1)
        def _():
            put_wait(a - 2, 0)
        transpose(0)
        @pl.when(a + 2 < BPW)
        def _():
            fetch(a + 2, 0)
        put(a, 0)
        @pl.when(a + 2 < BPW)
        def _():
            fetch_wait(a + 2, 0)

        gather_wait(a + 1, 1)
        @pl.when(p >= 1)
        def _():
            put_wait(a - 1, 1)
        transpose(1)
        @pl.when(a + 3 < BPW)
        def _():
            fetch(a + 3, 1)
        put(a + 1, 1)
        return 0

    lax.fori_loop(0, BPW // 2, body, 0)
    put_wait(BPW - 2, 0)
    put_wait(BPW - 1, 1)


def kernel(idx, table):
    idx_t = jnp.transpose(idx).reshape(-1).astype(jnp.int32)
    out_v = _emb_lookup(idx_t, table)
    return out_v.transpose(2, 4, 0, 1, 3).reshape(NI, NJ, D)


# speedup vs baseline: 1.4126x; 1.0116x over previous
import functools

import jax
import jax.numpy as jnp
from jax import lax
from jax.experimental import pallas as pl
from jax.experimental.pallas import tpu as pltpu
from jax.experimental.pallas import tpu_sc as plsc

CORPUS = 1_000_000
D = 64
NI, NJ = 16384, 50
B = NI * NJ
NW = 32
NIT = NI // 128
NBLK = NJ * NIT
BPW = NBLK // NW

_mesh = plsc.VectorSubcoreMesh(core_axis_name="c", subcore_axis_name="s")


@functools.partial(
    pl.kernel,
    out_type=jax.ShapeDtypeStruct((NJ, 8, 128, 8, 128), jnp.float32),
    mesh=_mesh,
    scratch_types=[
        pltpu.VMEM((2, 128), jnp.int32),
        pltpu.VMEM((2, 128, D), jnp.float32),
        pltpu.VMEM((2, 8, 8, 128), jnp.float32),
        pltpu.SemaphoreType.DMA,
        pltpu.SemaphoreType.DMA,
        pltpu.SemaphoreType.DMA,
        pltpu.SemaphoreType.DMA,
        pltpu.SemaphoreType.DMA,
        pltpu.SemaphoreType.DMA,
    ],
    compiler_params=pltpu.CompilerParams(
        use_tc_tiling_on_sc=False, needs_layout_passes=False
    ),
)
def _emb_lookup(idx_hbm, table_hbm, out_hbm, idx_v, rows_v, obuf_v,
                gi0, gi1, gr0, gr1, po0, po1):
    wid = lax.axis_index("s") * 2 + lax.axis_index("c")
    isem = (gi0, gi1)
    gsem = (gr0, gr1)
    osem = (po0, po1)

    def block_id(t):
        return wid * BPW + t

    def fetch(t, s):
        b = block_id(t)
        n0 = (b // NIT) * NI + (b % NIT) * 128
        pltpu.async_copy(idx_hbm.at[pl.ds(n0, 128)], idx_v.at[s], isem[s])

    def fetch_wait(t, s):
        b = block_id(t)
        n0 = (b // NIT) * NI + (b % NIT) * 128
        pltpu.make_async_copy(
            idx_hbm.at[pl.ds(n0, 128)], idx_v.at[s], isem[s]
        ).wait()
        pltpu.async_copy(table_hbm.at[idx_v.at[s]], rows_v.at[s], gsem[s])

    def gather_wait(t, s):
        pltpu.make_async_copy(
            table_hbm.at[idx_v.at[s]], rows_v.at[s], gsem[s]
        ).wait()

    def put(t, s):
        b = block_id(t)
        pltpu.async_copy(
            obuf_v.at[s], out_hbm.at[b // NIT, :, b % NIT], osem[s]
        )

    def put_wait(t, s):
        b = block_id(t)
        pltpu.make_async_copy(
            obuf_v.at[s], out_hbm.at[b // NIT, :, b % NIT], osem[s]
        ).wait()

    def transpose(s):
        rows = rows_v.at[s]
        obuf = obuf_v.at[s]
        lane = lax.iota(jnp.int32, 16)

        for q in range(8):
            rb = lane + (16 * q)

            def kbody(k, rb=rb, q=q):
                col = jax.lax.broadcast(k, (16,))
                obuf[k // 8, k % 8, pl.ds(16 * q, 16)] = plsc.load_gather(
                    rows, [rb, col]
                )

            plsc.parallel_loop(0, D, 1, unroll=16)(kbody)

    fetch(0, 0)
    fetch(1, 1)
    fetch_wait(0, 0)

    def body(p, _):
        a = 2 * p
        fetch_wait(a + 1, 1)

        gather_wait(a, 0)
        @pl.when(p >= 1)
        def _():
            put_wait(a - 2, 0)
        transpose(0)
        @pl.when(a + 2 < BPW)
        def _():
            fetch(a + 2, 0)
        put(a, 0)
        @pl.when(a + 2 < BPW)
        def _():
            fetch_wait(a + 2, 0)

        gather_wait(a + 1, 1)
        @pl.when(p >= 1)
        def _():
            put_wait(a - 1, 1)
        transpose(1)
        @pl.when(a + 3 < BPW)
        def _():
            fetch(a + 3, 1)
        put(a + 1, 1)
        return 0

    lax.fori_loop(0, BPW // 2, body, 0)
    put_wait(BPW - 2, 0)
    put_wait(BPW - 1, 1)


def kernel(idx, table):
    idx_t = jnp.transpose(idx).reshape(-1).astype(jnp.int32)
    out_v = _emb_lookup(idx_t, table)
    return out_v.transpose(2, 4, 0, 1, 3).reshape(NI, NJ, D)
